# baseline (speedup 1.0000x reference)
import functools

import jax
import jax.numpy as jnp
from jax import lax
from jax.experimental import pallas as pl
from jax.experimental.pallas import tpu as pltpu
from jax.experimental.pallas import tpu_sc as plsc

NC = 2
NS = 16
NW = NC * NS
BATCH = 128
LANES = 16


def _zero_rows(zero_v, shared, base, rows):
    nfull, rem = divmod(rows, BATCH)
    for t in range(nfull):
        pltpu.sync_copy(zero_v, shared.at[pl.ds(base + t * BATCH, BATCH)])
    if rem:
        pltpu.sync_copy(zero_v.at[pl.ds(0, rem)],
                        shared.at[pl.ds(base + nfull * BATCH, rem)])


def _deg_body(dst_hbm, deg_out, dst_v, deg_l, *, nb, n_pad):
    c = lax.axis_index("c")
    s = lax.axis_index("s")
    w = c * NS + s

    def zero(i, carry):
        deg_l[pl.ds(i * LANES, LANES)] = jnp.zeros((LANES,), jnp.float32)
        return carry
    lax.fori_loop(0, n_pad // LANES, zero, 0)

    pltpu.sync_copy(dst_hbm.at[w], dst_v)

    ones16 = jnp.ones((LANES,), jnp.float32)

    def scat(i, carry):
        j = i // (BATCH // LANES)
        q = i % (BATCH // LANES)
        idx = dst_v[j, pl.ds(q * LANES, LANES)]
        plsc.addupdate_scatter(deg_l, [idx], ones16)
        return carry
    lax.fori_loop(0, nb * (BATCH // LANES), scat, 0)

    pltpu.sync_copy(deg_l, deg_out.at[w])


def _agg_body(h_hbm, src_hbm, dst_hbm, part_out,
              dst_v, si0, si1, buf0, buf1, acc_sh,
              fi0, fi1, sg0, sg1, ss0, ss1, *, nb, rps):
    c = lax.axis_index("c")
    s = lax.axis_index("s")
    w = c * NS + s
    base = s * rps

    def fz(i, carry):
        for q in range(8):
            buf0[i, pl.ds(q * LANES, LANES)] = jnp.zeros((LANES,), jnp.float32)
        return carry
    lax.fori_loop(0, BATCH, fz, 0)

    _zero_rows(buf0, acc_sh, base, rps)
    pltpu.sync_copy(dst_hbm.at[w], dst_v)
    plsc.subcore_barrier()

    sidx = (si0, si1)
    bufs = (buf0, buf1)
    fis = (fi0, fi1)
    sgs = (sg0, sg1)
    sss = (ss0, ss1)

    def i_start(j, bi):
        pltpu.async_copy(src_hbm.at[w, pl.ds(j, 1)], sidx[bi], fis[bi])

    def i_wait(bi):
        pltpu.make_async_copy(src_hbm.at[w, pl.ds(0, 1)], sidx[bi], fis[bi]).wait()

    def g_start(bi):
        pltpu.async_copy(h_hbm.at[sidx[bi].at[0]], bufs[bi], sgs[bi])

    def g_wait(bi):
        pltpu.make_async_copy(h_hbm.at[sidx[bi].at[0]], bufs[bi], sgs[bi]).wait()

    def s_start(j, bi):
        pltpu.async_copy(bufs[bi], acc_sh.at[dst_v.at[j]], sss[bi], add=True)

    def s_wait(bi):
        pltpu.make_async_copy(bufs[bi], acc_sh.at[dst_v.at[0]], sss[bi]).wait()

    i_start(0, 0)
    i_start(1, 1)
    for j in (0, 1):
        i_wait(j)
        g_start(j)
        g_wait(j)
        i_start(j + 2, j)
        s_start(j, j)

    def pair(k, carry):
        j0 = 2 * k
        for bi in (0, 1):
            j = j0 + bi
            i_wait(bi)
            s_wait(bi)
            g_start(bi)
            g_wait(bi)
            i_start(j + 2, bi)
            s_start(j, bi)
        return carry
    npair_hi = (nb - 3) // 2
    lax.fori_loop(1, npair_hi, pair, 0)

    for j in range(2 * npair_hi, nb):
        bi = j % 2
        i_wait(bi)
        s_wait(bi)
        g_start(bi)
        g_wait(bi)
        if j + 2 < nb:
            i_start(j + 2, bi)
        s_start(j, bi)
    s_wait((nb - 2) % 2)
    s_wait((nb - 1) % 2)

    plsc.subcore_barrier()
    pltpu.sync_copy(acc_sh.at[pl.ds(base, rps)], part_out.at[c, pl.ds(base, rps)])


def _sc_mesh():
    return plsc.VectorSubcoreMesh(core_axis_name="c", subcore_axis_name="s")


def _deg_call(n_pad, nb):
    return pl.kernel(
        functools.partial(_deg_body, nb=nb, n_pad=n_pad),
        out_type=jax.ShapeDtypeStruct((NW, n_pad), jnp.float32),
        mesh=_sc_mesh(),
        compiler_params=pltpu.CompilerParams(needs_layout_passes=False),
        scratch_types=[
            pltpu.VMEM((nb, BATCH), jnp.int32),
            pltpu.VMEM((n_pad,), jnp.float32),
        ],
    )


def _agg_call(n_pad, nb, d_out):
    return pl.kernel(
        functools.partial(_agg_body, nb=nb, rps=n_pad // NS),
        out_type=jax.ShapeDtypeStruct((NC, n_pad, d_out), jnp.float32),
        mesh=_sc_mesh(),
        scratch_types=[
            pltpu.VMEM((nb, BATCH), jnp.int32),
            pltpu.VMEM((1, BATCH), jnp.int32),
            pltpu.VMEM((1, BATCH), jnp.int32),
            pltpu.VMEM((BATCH, d_out), jnp.float32),
            pltpu.VMEM((BATCH, d_out), jnp.float32),
            pltpu.VMEM_SHARED((n_pad, d_out), jnp.float32),
            pltpu.SemaphoreType.DMA,
            pltpu.SemaphoreType.DMA,
            pltpu.SemaphoreType.DMA,
            pltpu.SemaphoreType.DMA,
            pltpu.SemaphoreType.DMA,
            pltpu.SemaphoreType.DMA,
        ],
    )


def _pre_body(x_ref, w_ref, degp_ref, h_ref, isd_ref):
    deg = jnp.sum(degp_ref[...], axis=0)
    isd = lax.rsqrt(jnp.maximum(deg, 1.0))
    isd_ref[...] = isd
    h = jnp.dot(x_ref[...], w_ref[...], preferred_element_type=jnp.float32)
    h_ref[...] = h * isd[:, None]


def _post_body(parts_ref, isd_ref, b_ref, y_ref):
    acc = parts_ref[0] + parts_ref[1]
    y_ref[...] = acc * isd_ref[...][:, None] + b_ref[...][None, :]


def kernel(x, edge_index, W, b):
    n, d_in = x.shape
    d_out = W.shape[1]
    e = edge_index.shape[1]
    assert n % NS == 0 and d_in % LANES == 0

    nb = -(-e // (NW * BATCH))
    assert nb >= 3
    e_pad = NW * nb * BATCH
    n_pad = -(-(n + NS) // (NS * 8)) * (NS * 8)
    rps = n_pad // NS

    src = edge_index[0]
    dst = edge_index[1]
    pad = e_pad - e
    if pad:
        extra = jnp.arange(pad, dtype=jnp.int32)
        src = jnp.concatenate([src, extra % NS])
        dst = jnp.concatenate([dst, n + (extra % NS)])
    src3 = src.reshape(NW, nb, BATCH)
    dst3 = dst.reshape(NW, nb, BATCH)

    deg_parts = _deg_call(n_pad, nb)(dst3)

    blk = 1024
    nblk = -(-n // blk)
    h, isd = pl.pallas_call(
        _pre_body,
        grid=(nblk,),
        in_specs=[
            pl.BlockSpec((blk, d_in), lambda i: (i, 0)),
            pl.BlockSpec((d_in, d_out), lambda i: (0, 0)),
            pl.BlockSpec((NW, blk), lambda i: (0, i)),
        ],
        out_specs=[
            pl.BlockSpec((blk, d_out), lambda i: (i, 0)),
            pl.BlockSpec((blk,), lambda i: (i,)),
        ],
        out_shape=[
            jax.ShapeDtypeStruct((n, d_out), jnp.float32),
            jax.ShapeDtypeStruct((n_pad,), jnp.float32),
        ],
    )(x, W, deg_parts)

    parts = _agg_call(n_pad, nb, d_out)(h, src3, dst3)

    y = pl.pallas_call(
        _post_body,
        grid=(nblk,),
        in_specs=[
            pl.BlockSpec((NC, blk, d_out), lambda i: (0, i, 0)),
            pl.BlockSpec((blk,), lambda i: (i,)),
            pl.BlockSpec((d_out,), lambda i: (0,)),
        ],
        out_specs=pl.BlockSpec((blk, d_out), lambda i: (i, 0)),
        out_shape=jax.ShapeDtypeStruct((n, d_out), jnp.float32),
    )(parts, isd, b)
    return y


# speedup vs baseline: 1.0171x; 1.0171x over previous
import functools

import jax
import jax.numpy as jnp
from jax import lax
from jax.experimental import pallas as pl
from jax.experimental.pallas import tpu as pltpu
from jax.experimental.pallas import tpu_sc as plsc

NC = 2
NS = 16
NW = NC * NS
BATCH = 128
LANES = 16


def _zero_rows(zero_v, shared, base, rows):
    nfull, rem = divmod(rows, BATCH)
    for t in range(nfull):
        pltpu.sync_copy(zero_v, shared.at[pl.ds(base + t * BATCH, BATCH)])
    if rem:
        pltpu.sync_copy(zero_v.at[pl.ds(0, rem)],
                        shared.at[pl.ds(base + nfull * BATCH, rem)])


def _deg_body(dst_hbm, deg_out, dst_v, deg_l, *, nb, n_pad):
    c = lax.axis_index("c")
    s = lax.axis_index("s")
    w = c * NS + s

    def zero(i, carry):
        deg_l[pl.ds(i * LANES, LANES)] = jnp.zeros((LANES,), jnp.float32)
        return carry
    lax.fori_loop(0, n_pad // LANES, zero, 0)

    pltpu.sync_copy(dst_hbm.at[w], dst_v)

    ones16 = jnp.ones((LANES,), jnp.float32)

    def scat(i, carry):
        j = i // (BATCH // LANES)
        q = i % (BATCH // LANES)
        idx = dst_v[j, pl.ds(q * LANES, LANES)]
        plsc.addupdate_scatter(deg_l, [idx], ones16)
        return carry
    lax.fori_loop(0, nb * (BATCH // LANES), scat, 0)

    pltpu.sync_copy(deg_l, deg_out.at[w])


def _agg_body(h_hbm, src_hbm, dst_hbm, part_out,
              dst_v, si0, si1, buf0, buf1, acc_sh,
              fi0, fi1, sg0, sg1, ss0, ss1, *, nb, rps):
    c = lax.axis_index("c")
    s = lax.axis_index("s")
    w = c * NS + s
    base = s * rps

    def fz(i, carry):
        for q in range(8):
            buf0[i, pl.ds(q * LANES, LANES)] = jnp.zeros((LANES,), jnp.float32)
        return carry
    lax.fori_loop(0, BATCH, fz, 0)

    _zero_rows(buf0, acc_sh, base, rps)
    pltpu.sync_copy(dst_hbm.at[w], dst_v)
    plsc.subcore_barrier()

    sidx = (si0, si1)
    bufs = (buf0, buf1)
    fis = (fi0, fi1)
    sgs = (sg0, sg1)
    sss = (ss0, ss1)

    def i_start(k, bi):
        pltpu.async_copy(src_hbm.at[w, pl.ds(8 * k, 8)], sidx[bi], fis[bi])

    def i_wait(bi):
        pltpu.make_async_copy(src_hbm.at[w, pl.ds(0, 8)], sidx[bi], fis[bi]).wait()

    def g_start(r, ib):
        pltpu.async_copy(h_hbm.at[sidx[ib].at[r]], bufs[r % 2], sgs[r % 2])

    def g_wait(bi):
        pltpu.make_async_copy(h_hbm.at[sidx[0].at[0]], bufs[bi], sgs[bi]).wait()

    def s_start(j, bi):
        pltpu.async_copy(bufs[bi], acc_sh.at[dst_v.at[j]], sss[bi], add=True)

    def s_wait(bi):
        pltpu.make_async_copy(bufs[bi], acc_sh.at[dst_v.at[0]], sss[bi]).wait()

    nblocks = nb // 8

    def block(k, ib, first):
        i_wait(ib)
        for r in range(8):
            j = 8 * k + r
            br = r % 2
            if not (first and r < 2):
                s_wait(br)
            g_start(r, ib)
            g_wait(br)
            s_start(j, br)

    i_start(0, 0)
    i_start(1, 1)
    block(0, 0, True)
    i_start(2, 0)

    npair = (nblocks - 4) // 2

    def fbody(m, carry):
        k0 = 2 * m + 1
        block(k0, 1, False)
        i_start(k0 + 2, 1)
        block(k0 + 1, 0, False)
        i_start(k0 + 3, 0)
        return carry
    lax.fori_loop(0, npair, fbody, 0)

    for k in range(2 * npair + 1, nblocks):
        block(k, k % 2, False)
        if k + 2 < nblocks:
            i_start(k + 2, k % 2)
    s_wait(0)
    s_wait(1)

    plsc.subcore_barrier()
    pltpu.sync_copy(acc_sh.at[pl.ds(base, rps)], part_out.at[c, pl.ds(base, rps)])


def _sc_mesh():
    return plsc.VectorSubcoreMesh(core_axis_name="c", subcore_axis_name="s")


def _deg_call(n_pad, nb):
    return pl.kernel(
        functools.partial(_deg_body, nb=nb, n_pad=n_pad),
        out_type=jax.ShapeDtypeStruct((NW, n_pad), jnp.float32),
        mesh=_sc_mesh(),
        compiler_params=pltpu.CompilerParams(needs_layout_passes=False),
        scratch_types=[
            pltpu.VMEM((nb, BATCH), jnp.int32),
            pltpu.VMEM((n_pad,), jnp.float32),
        ],
    )


def _agg_call(n_pad, nb, d_out):
    return pl.kernel(
        functools.partial(_agg_body, nb=nb, rps=n_pad // NS),
        out_type=jax.ShapeDtypeStruct((NC, n_pad, d_out), jnp.float32),
        mesh=_sc_mesh(),
        scratch_types=[
            pltpu.VMEM((nb, BATCH), jnp.int32),
            pltpu.VMEM((8, BATCH), jnp.int32),
            pltpu.VMEM((8, BATCH), jnp.int32),
            pltpu.VMEM((BATCH, d_out), jnp.float32),
            pltpu.VMEM((BATCH, d_out), jnp.float32),
            pltpu.VMEM_SHARED((n_pad, d_out), jnp.float32),
            pltpu.SemaphoreType.DMA,
            pltpu.SemaphoreType.DMA,
            pltpu.SemaphoreType.DMA,
            pltpu.SemaphoreType.DMA,
            pltpu.SemaphoreType.DMA,
            pltpu.SemaphoreType.DMA,
        ],
    )


def _pre_body(x_ref, w_ref, degp_ref, h_ref, isd_ref, *, n):
    deg = jnp.sum(degp_ref[...], axis=0)
    isd = lax.rsqrt(jnp.maximum(deg, 1.0))
    isd_ref[...] = isd
    h = jnp.dot(x_ref[...], w_ref[...], preferred_element_type=jnp.float32)
    h_ref[...] = h * isd[:n, None]


def _post_body(parts_ref, isd_ref, b_ref, y_ref, *, n):
    acc = parts_ref[0, :n, :] + parts_ref[1, :n, :]
    y_ref[...] = acc * isd_ref[...][:n, None] + b_ref[...][None, :]


def kernel(x, edge_index, W, b):
    n, d_in = x.shape
    d_out = W.shape[1]
    e = edge_index.shape[1]
    assert n % NS == 0 and d_in % LANES == 0

    nb = -(-(-(-e // (NW * BATCH))) // 8) * 8
    assert nb >= 16
    e_pad = NW * nb * BATCH
    n_pad = -(-(n + NS) // (NS * 8)) * (NS * 8)
    rps = n_pad // NS

    src = edge_index[0]
    dst = edge_index[1]
    pad = e_pad - e
    if pad:
        extra = jnp.arange(pad, dtype=jnp.int32)
        src = jnp.concatenate([src, extra % NS])
        dst = jnp.concatenate([dst, n + (extra % NS)])
    src3 = src.reshape(NW, nb, BATCH)
    dst3 = dst.reshape(NW, nb, BATCH)

    deg_parts = _deg_call(n_pad, nb)(dst3)

    h, isd = pl.pallas_call(
        functools.partial(_pre_body, n=n),
        out_shape=[
            jax.ShapeDtypeStruct((n, d_out), jnp.float32),
            jax.ShapeDtypeStruct((n_pad,), jnp.float32),
        ],
    )(x, W, deg_parts)

    parts = _agg_call(n_pad, nb, d_out)(h, src3, dst3)

    y = pl.pallas_call(
        functools.partial(_post_body, n=n),
        out_shape=jax.ShapeDtypeStruct((n, d_out), jnp.float32),
    )(parts, isd, b)
    return y


# speedup vs baseline: 1.0375x; 1.0201x over previous
import functools

import jax
import jax.numpy as jnp
from jax import lax
from jax.experimental import pallas as pl
from jax.experimental.pallas import tpu as pltpu
from jax.experimental.pallas import tpu_sc as plsc

NC = 2
NS = 16
NW = NC * NS
BATCH = 128
LANES = 16


def _zero_rows(zero_v, shared, base, rows):
    nfull, rem = divmod(rows, BATCH)
    for t in range(nfull):
        pltpu.sync_copy(zero_v, shared.at[pl.ds(base + t * BATCH, BATCH)])
    if rem:
        pltpu.sync_copy(zero_v.at[pl.ds(0, rem)],
                        shared.at[pl.ds(base + nfull * BATCH, rem)])


def _deg_body(dst_hbm, deg_out, dst_v, deg_l, *, nb, n_pad):
    c = lax.axis_index("c")
    s = lax.axis_index("s")
    w = c * NS + s

    def zero(i, carry):
        deg_l[pl.ds(i * LANES, LANES)] = jnp.zeros((LANES,), jnp.float32)
        return carry
    lax.fori_loop(0, n_pad // LANES, zero, 0)

    pltpu.sync_copy(dst_hbm.at[w], dst_v)

    ones16 = jnp.ones((LANES,), jnp.float32)

    def scat(i, carry):
        j = i // (BATCH // LANES)
        q = i % (BATCH // LANES)
        idx = dst_v[j, pl.ds(q * LANES, LANES)]
        plsc.addupdate_scatter(deg_l, [idx], ones16)
        return carry
    lax.fori_loop(0, nb * (BATCH // LANES), scat, 0)

    pltpu.sync_copy(deg_l, deg_out.at[w])


def _agg_body(h_hbm, src_hbm, dst_hbm, part_out,
              dst_v, si0, si1, buf0, buf1, acc_sh,
              fi0, fi1, sg0, sg1, ss0, ss1, *, nb, rps):
    c = lax.axis_index("c")
    s = lax.axis_index("s")
    w = c * NS + s
    base = s * rps

    def fz(i, carry):
        for q in range(8):
            buf0[i, pl.ds(q * LANES, LANES)] = jnp.zeros((LANES,), jnp.float32)
        return carry
    lax.fori_loop(0, BATCH, fz, 0)

    _zero_rows(buf0, acc_sh, base, rps)
    pltpu.sync_copy(dst_hbm.at[w], dst_v)
    plsc.subcore_barrier()

    sidx = (si0, si1)
    bufs = (buf0, buf1)
    fis = (fi0, fi1)
    sgs = (sg0, sg1)
    sss = (ss0, ss1)

    def i_start(j, bi):
        pltpu.async_copy(src_hbm.at[w, pl.ds(j, 1)], sidx[bi], fis[bi])

    def i_wait(bi):
        pltpu.make_async_copy(src_hbm.at[w, pl.ds(0, 1)], sidx[bi], fis[bi]).wait()

    def g_start(bi):
        pltpu.async_copy(h_hbm.at[sidx[bi].at[0]], bufs[bi], sgs[bi])

    def g_wait(bi):
        pltpu.make_async_copy(h_hbm.at[sidx[bi].at[0]], bufs[bi], sgs[bi]).wait()

    def s_start(j, bi):
        pltpu.async_copy(bufs[bi], acc_sh.at[dst_v.at[j]], sss[bi], add=True)

    def s_wait(bi):
        pltpu.make_async_copy(bufs[bi], acc_sh.at[dst_v.at[0]], sss[bi]).wait()

    i_start(0, 0)
    i_start(1, 1)
    for j in (0, 1):
        i_wait(j)
        g_start(j)
        g_wait(j)
        i_start(j + 2, j)
        s_start(j, j)

    def pair(k, carry):
        j0 = 2 * k
        for bi in (0, 1):
            j = j0 + bi
            i_wait(bi)
            s_wait(bi)
            g_start(bi)
            g_wait(bi)
            i_start(j + 2, bi)
            s_start(j, bi)
        return carry
    npair_hi = (nb - 3) // 2
    lax.fori_loop(1, npair_hi, pair, 0)

    for j in range(2 * npair_hi, nb):
        bi = j % 2
        i_wait(bi)
        s_wait(bi)
        g_start(bi)
        g_wait(bi)
        if j + 2 < nb:
            i_start(j + 2, bi)
        s_start(j, bi)
    s_wait((nb - 2) % 2)
    s_wait((nb - 1) % 2)

    plsc.subcore_barrier()
    pltpu.sync_copy(acc_sh.at[pl.ds(base, rps)], part_out.at[c, pl.ds(base, rps)])


def _sc_mesh():
    return plsc.VectorSubcoreMesh(core_axis_name="c", subcore_axis_name="s")


def _deg_call(n_pad, nb):
    return pl.kernel(
        functools.partial(_deg_body, nb=nb, n_pad=n_pad),
        out_type=jax.ShapeDtypeStruct((NW, n_pad), jnp.float32),
        mesh=_sc_mesh(),
        compiler_params=pltpu.CompilerParams(needs_layout_passes=False),
        scratch_types=[
            pltpu.VMEM((nb, BATCH), jnp.int32),
            pltpu.VMEM((n_pad,), jnp.float32),
        ],
    )


def _agg_call(n_pad, nb, d_out):
    return pl.kernel(
        functools.partial(_agg_body, nb=nb, rps=n_pad // NS),
        out_type=jax.ShapeDtypeStruct((NC, n_pad, d_out), jnp.float32),
        mesh=_sc_mesh(),
        scratch_types=[
            pltpu.VMEM((nb, BATCH), jnp.int32),
            pltpu.VMEM((1, BATCH), jnp.int32),
            pltpu.VMEM((1, BATCH), jnp.int32),
            pltpu.VMEM((BATCH, d_out), jnp.float32),
            pltpu.VMEM((BATCH, d_out), jnp.float32),
            pltpu.VMEM_SHARED((n_pad, d_out), jnp.float32),
            pltpu.SemaphoreType.DMA,
            pltpu.SemaphoreType.DMA,
            pltpu.SemaphoreType.DMA,
            pltpu.SemaphoreType.DMA,
            pltpu.SemaphoreType.DMA,
            pltpu.SemaphoreType.DMA,
        ],
    )


def _mm_body(x_ref, w_ref, xw_ref):
    xw_ref[...] = jnp.dot(
        x_ref[...], w_ref[...], preferred_element_type=jnp.float32)


def _pre_body(xw_ref, degp_ref, h_ref, isd_ref, *, n):
    deg = jnp.sum(degp_ref[...], axis=0)
    isd = lax.rsqrt(jnp.maximum(deg, 1.0))
    isd_ref[...] = isd
    h_ref[...] = xw_ref[...] * isd[:n, None]


def _post_body(parts_ref, isd_ref, b_ref, y_ref, *, n):
    acc = parts_ref[0, :n, :] + parts_ref[1, :n, :]
    y_ref[...] = acc * isd_ref[...][:n, None] + b_ref[...][None, :]


def kernel(x, edge_index, W, b):
    n, d_in = x.shape
    d_out = W.shape[1]
    e = edge_index.shape[1]
    assert n % NS == 0 and d_in % LANES == 0

    nb = -(-e // (NW * BATCH))
    assert nb >= 3
    e_pad = NW * nb * BATCH
    n_pad = -(-(n + NS) // (NS * 8)) * (NS * 8)
    rps = n_pad // NS

    src = edge_index[0]
    dst = edge_index[1]
    pad = e_pad - e
    if pad:
        extra = jnp.arange(pad, dtype=jnp.int32)
        src = jnp.concatenate([src, extra % NS])
        dst = jnp.concatenate([dst, n + (extra % NS)])
    src3 = src.reshape(NW, nb, BATCH)
    dst3 = dst.reshape(NW, nb, BATCH)

    xw = pl.pallas_call(
        _mm_body,
        out_shape=jax.ShapeDtypeStruct((n, d_out), jnp.float32),
    )(x, W)
    deg_parts = _deg_call(n_pad, nb)(dst3)

    h, isd = pl.pallas_call(
        functools.partial(_pre_body, n=n),
        out_shape=[
            jax.ShapeDtypeStruct((n, d_out), jnp.float32),
            jax.ShapeDtypeStruct((n_pad,), jnp.float32),
        ],
    )(xw, deg_parts)

    parts = _agg_call(n_pad, nb, d_out)(h, src3, dst3)

    y = pl.pallas_call(
        functools.partial(_post_body, n=n),
        out_shape=jax.ShapeDtypeStruct((n, d_out), jnp.float32),
    )(parts, isd, b)
    return y


# speedup vs baseline: 1.0383x; 1.0008x over previous
import functools

import jax
import jax.numpy as jnp
from jax import lax
from jax.experimental import pallas as pl
from jax.experimental.pallas import tpu as pltpu
from jax.experimental.pallas import tpu_sc as plsc

NC = 2
NS = 16
NW = NC * NS
BATCH = 128
LANES = 16


def _zero_rows(zero_v, shared, base, rows):
    nfull, rem = divmod(rows, BATCH)
    for t in range(nfull):
        pltpu.sync_copy(zero_v, shared.at[pl.ds(base + t * BATCH, BATCH)])
    if rem:
        pltpu.sync_copy(zero_v.at[pl.ds(0, rem)],
                        shared.at[pl.ds(base + nfull * BATCH, rem)])


def _deg_body(dst_hbm, deg_out, dst_v, deg_l, *, nb, n_pad):
    c = lax.axis_index("c")
    s = lax.axis_index("s")
    w = c * NS + s

    def zero(i, carry):
        deg_l[pl.ds(i * LANES, LANES)] = jnp.zeros((LANES,), jnp.float32)
        return carry
    lax.fori_loop(0, n_pad // LANES, zero, 0)

    pltpu.sync_copy(dst_hbm.at[w], dst_v)

    ones16 = jnp.ones((LANES,), jnp.float32)

    def scat(j, carry):
        for q in range(BATCH // LANES):
            idx = dst_v[j, pl.ds(q * LANES, LANES)]
            plsc.addupdate_scatter(deg_l, [idx], ones16)
        return carry
    lax.fori_loop(0, nb, scat, 0)

    pltpu.sync_copy(deg_l, deg_out.at[w])


def _agg_body(h_hbm, src_hbm, dst_hbm, part_out,
              dst_v, si0, si1, buf0, buf1, acc_sh,
              fi0, fi1, sg0, sg1, ss0, ss1, *, nb, rps):
    c = lax.axis_index("c")
    s = lax.axis_index("s")
    w = c * NS + s
    base = s * rps

    def fz(i, carry):
        for q in range(8):
            buf0[i, pl.ds(q * LANES, LANES)] = jnp.zeros((LANES,), jnp.float32)
        return carry
    lax.fori_loop(0, BATCH, fz, 0)

    _zero_rows(buf0, acc_sh, base, rps)
    pltpu.sync_copy(dst_hbm.at[w], dst_v)
    plsc.subcore_barrier()

    sidx = (si0, si1)
    bufs = (buf0, buf1)
    fis = (fi0, fi1)
    sgs = (sg0, sg1)
    sss = (ss0, ss1)

    def i_start(j, bi):
        pltpu.async_copy(src_hbm.at[w, pl.ds(j, 1)], sidx[bi], fis[bi])

    def i_wait(bi):
        pltpu.make_async_copy(src_hbm.at[w, pl.ds(0, 1)], sidx[bi], fis[bi]).wait()

    def g_start(bi):
        pltpu.async_copy(h_hbm.at[sidx[bi].at[0]], bufs[bi], sgs[bi])

    def g_wait(bi):
        pltpu.make_async_copy(h_hbm.at[sidx[bi].at[0]], bufs[bi], sgs[bi]).wait()

    def s_start(j, bi):
        pltpu.async_copy(bufs[bi], acc_sh.at[dst_v.at[j]], sss[bi], add=True)

    def s_wait(bi):
        pltpu.make_async_copy(bufs[bi], acc_sh.at[dst_v.at[0]], sss[bi]).wait()

    i_start(0, 0)
    i_start(1, 1)
    for j in (0, 1):
        i_wait(j)
        g_start(j)
        g_wait(j)
        i_start(j + 2, j)
        s_start(j, j)

    def pair(k, carry):
        j0 = 2 * k
        for bi in (0, 1):
            j = j0 + bi
            i_wait(bi)
            s_wait(bi)
            g_start(bi)
            g_wait(bi)
            i_start(j + 2, bi)
            s_start(j, bi)
        return carry
    npair_hi = (nb - 3) // 2
    lax.fori_loop(1, npair_hi, pair, 0)

    for j in range(2 * npair_hi, nb):
        bi = j % 2
        i_wait(bi)
        s_wait(bi)
        g_start(bi)
        g_wait(bi)
        if j + 2 < nb:
            i_start(j + 2, bi)
        s_start(j, bi)
    s_wait((nb - 2) % 2)
    s_wait((nb - 1) % 2)

    plsc.subcore_barrier()
    pltpu.sync_copy(acc_sh.at[pl.ds(base, rps)], part_out.at[c, pl.ds(base, rps)])


def _sc_mesh():
    return plsc.VectorSubcoreMesh(core_axis_name="c", subcore_axis_name="s")


def _deg_call(n_pad, nb):
    return pl.kernel(
        functools.partial(_deg_body, nb=nb, n_pad=n_pad),
        out_type=jax.ShapeDtypeStruct((NW, n_pad), jnp.float32),
        mesh=_sc_mesh(),
        compiler_params=pltpu.CompilerParams(needs_layout_passes=False),
        scratch_types=[
            pltpu.VMEM((nb, BATCH), jnp.int32),
            pltpu.VMEM((n_pad,), jnp.float32),
        ],
    )


def _agg_call(n_pad, nb, d_out):
    return pl.kernel(
        functools.partial(_agg_body, nb=nb, rps=n_pad // NS),
        out_type=jax.ShapeDtypeStruct((NC, n_pad, d_out), jnp.float32),
        mesh=_sc_mesh(),
        scratch_types=[
            pltpu.VMEM((nb, BATCH), jnp.int32),
            pltpu.VMEM((1, BATCH), jnp.int32),
            pltpu.VMEM((1, BATCH), jnp.int32),
            pltpu.VMEM((BATCH, d_out), jnp.float32),
            pltpu.VMEM((BATCH, d_out), jnp.float32),
            pltpu.VMEM_SHARED((n_pad, d_out), jnp.float32),
            pltpu.SemaphoreType.DMA,
            pltpu.SemaphoreType.DMA,
            pltpu.SemaphoreType.DMA,
            pltpu.SemaphoreType.DMA,
            pltpu.SemaphoreType.DMA,
            pltpu.SemaphoreType.DMA,
        ],
    )


def _mm_body(x_ref, w_ref, xw_ref):
    xw_ref[...] = jnp.dot(
        x_ref[...], w_ref[...], preferred_element_type=jnp.float32)


def _pre_body(xw_ref, degp_ref, h_ref, isd_ref, *, n):
    deg = jnp.sum(degp_ref[...], axis=0)
    isd = lax.rsqrt(jnp.maximum(deg, 1.0))
    isd_ref[...] = isd
    h_ref[...] = xw_ref[...] * isd[:n, None]


def _post_body(parts_ref, isd_ref, b_ref, y_ref, *, n):
    acc = parts_ref[0, :n, :] + parts_ref[1, :n, :]
    y_ref[...] = acc * isd_ref[...][:n, None] + b_ref[...][None, :]


def kernel(x, edge_index, W, b):
    n, d_in = x.shape
    d_out = W.shape[1]
    e = edge_index.shape[1]
    assert n % NS == 0 and d_in % LANES == 0

    nb = -(-e // (NW * BATCH))
    assert nb >= 3
    e_pad = NW * nb * BATCH
    n_pad = -(-(n + NS) // (NS * 8)) * (NS * 8)
    rps = n_pad // NS

    src = edge_index[0]
    dst = edge_index[1]
    pad = e_pad - e
    if pad:
        extra = jnp.arange(pad, dtype=jnp.int32)
        src = jnp.concatenate([src, extra % NS])
        dst = jnp.concatenate([dst, n + (extra % NS)])
    src3 = src.reshape(NW, nb, BATCH)
    dst3 = dst.reshape(NW, nb, BATCH)

    xw = pl.pallas_call(
        _mm_body,
        out_shape=jax.ShapeDtypeStruct((n, d_out), jnp.float32),
    )(x, W)
    deg_parts = _deg_call(n_pad, nb)(dst3)

    h, isd = pl.pallas_call(
        functools.partial(_pre_body, n=n),
        out_shape=[
            jax.ShapeDtypeStruct((n, d_out), jnp.float32),
            jax.ShapeDtypeStruct((n_pad,), jnp.float32),
        ],
    )(xw, deg_parts)

    parts = _agg_call(n_pad, nb, d_out)(h, src3, dst3)

    y = pl.pallas_call(
        functools.partial(_post_body, n=n),
        out_shape=jax.ShapeDtypeStruct((n, d_out), jnp.float32),
    )(parts, isd, b)
    return y
